# baseline (speedup 1.0000x reference)
import functools

import jax
import jax.numpy as jnp
import numpy as np
from jax import lax
from jax.experimental import pallas as pl
from jax.experimental.pallas import tpu as pltpu
from jax.experimental.pallas import tpu_sc as plsc

_NUM = 16


@functools.lru_cache(maxsize=None)
def _pools(h, w):
    with jax.ensure_compile_time_eval():
        with jax.default_device(jax.devices("cpu")[0]):
            kx = jax.random.fold_in(jax.random.key(1), 0)
            ky = jax.random.fold_in(jax.random.key(1), 1)
            pool_x = np.asarray(jax.random.permutation(kx, h))
            pool_y = np.asarray(jax.random.permutation(ky, w))
    return pool_x, pool_y


def _make_sc_kernel(b, c, h, w, planes_per_worker):
    mesh = plsc.VectorSubcoreMesh(core_axis_name="c", subcore_axis_name="s")
    nc = plsc.get_sparse_core_info().num_cores
    pw = planes_per_worker

    @functools.partial(
        pl.kernel,
        out_type=jax.ShapeDtypeStruct((b, c, h, w), jnp.float32),
        mesh=mesh,
        compiler_params=pltpu.CompilerParams(
            needs_layout_passes=False, skip_device_barrier=True),
        scratch_types=[
            pltpu.VMEM((h, w), jnp.float32),
            pltpu.VMEM((h, w), jnp.float32),
            pltpu.VMEM((4, _NUM), jnp.int32),
            pltpu.SemaphoreType.DMA,
            pltpu.SemaphoreType.DMA,
            pltpu.SemaphoreType.DMA,
            pltpu.SemaphoreType.DMA,
        ],
    )
    def sc_kernel(x_hbm, idx_hbm, out_hbm, buf0, buf1, idx_v,
                  rsem0, rsem1, wsem0, wsem1):
        wid = lax.axis_index("s") * nc + lax.axis_index("c")
        base = wid * pw
        pltpu.sync_copy(idx_hbm, idx_v)
        pxv = idx_v[0, :]
        pyv = idx_v[1, :]
        sxv = idx_v[2, :]
        syv = idx_v[3, :]
        bufs = (buf0, buf1)
        rsems = (rsem0, rsem1)
        wsems = (wsem0, wsem1)
        reads = [None, None]
        writes = [None, None]
        for p in range(pw + 1):
            if p < pw:
                k = p % 2
                if writes[k] is not None:
                    writes[k].wait()
                pi = base + p
                reads[k] = pltpu.make_async_copy(
                    x_hbm.at[pi // c, pi % c], bufs[k], rsems[k])
                reads[k].start()
            if p >= 1:
                k = (p - 1) % 2
                reads[k].wait()
                vals = plsc.load_gather(bufs[k], [sxv, syv])
                plsc.store_scatter(bufs[k], [pxv, pyv], vals)
                pi = base + p - 1
                writes[k] = pltpu.make_async_copy(
                    bufs[k], out_hbm.at[pi // c, pi % c], wsems[k])
                writes[k].start()
        writes[(pw - 1) % 2].wait()
        writes[pw % 2].wait()

    return sc_kernel


def kernel(x):
    b, c, h, w = x.shape
    pool_x, pool_y = _pools(h, w)
    px = jnp.asarray(pool_x[:_NUM])
    py = jnp.asarray(pool_y[:_NUM])
    sx = pool_x[-_NUM:]
    sy = pool_y[-_NUM:]
    idx = jnp.asarray(
        np.stack([pool_x[:_NUM], pool_y[:_NUM], sx, sy]).astype(np.int32))

    p = b * c
    info = plsc.get_sparse_core_info()
    nw = info.num_cores * info.num_subcores
    assert p % nw == 0
    out = _make_sc_kernel(b, c, h, w, p // nw)(x, idx)
    return out, (px, py)


# speedup vs baseline: 1.0024x; 1.0024x over previous
import functools

import jax
import jax.numpy as jnp
import numpy as np
from jax import lax
from jax.experimental import pallas as pl
from jax.experimental.pallas import tpu as pltpu
from jax.experimental.pallas import tpu_sc as plsc

_NUM = 16


@functools.lru_cache(maxsize=None)
def _pools(h, w):
    with jax.ensure_compile_time_eval():
        with jax.default_device(jax.devices("cpu")[0]):
            kx = jax.random.fold_in(jax.random.key(1), 0)
            ky = jax.random.fold_in(jax.random.key(1), 1)
            pool_x = np.asarray(jax.random.permutation(kx, h))
            pool_y = np.asarray(jax.random.permutation(ky, w))
    return pool_x, pool_y


def _make_sc_kernel(b, c, h, w, planes_per_worker):
    mesh = plsc.VectorSubcoreMesh(core_axis_name="c", subcore_axis_name="s")
    nc = plsc.get_sparse_core_info().num_cores
    pw = planes_per_worker

    @functools.partial(
        pl.kernel,
        out_type=jax.ShapeDtypeStruct((b, c, h, w), jnp.float32),
        mesh=mesh,
        compiler_params=pltpu.CompilerParams(needs_layout_passes=False),
        scratch_types=[
            pltpu.VMEM((h, w), jnp.float32),
            pltpu.VMEM((h, w), jnp.float32),
            pltpu.VMEM((4, _NUM), jnp.int32),
            pltpu.SemaphoreType.DMA,
            pltpu.SemaphoreType.DMA,
            pltpu.SemaphoreType.DMA,
            pltpu.SemaphoreType.DMA,
        ],
    )
    def sc_kernel(x_hbm, idx_hbm, out_hbm, buf0, buf1, idx_v,
                  rsem0, rsem1, wsem0, wsem1):
        wid = lax.axis_index("s") * nc + lax.axis_index("c")
        base = wid * pw
        pltpu.sync_copy(idx_hbm, idx_v)
        pxv = idx_v[0, :]
        pyv = idx_v[1, :]
        sxv = idx_v[2, :]
        syv = idx_v[3, :]
        bufs = (buf0, buf1)
        rsems = (rsem0, rsem1)
        wsems = (wsem0, wsem1)
        reads = [None, None]
        writes = [None, None]
        for p in range(pw + 1):
            if p < pw:
                k = p % 2
                if writes[k] is not None:
                    writes[k].wait()
                pi = base + p
                reads[k] = pltpu.make_async_copy(
                    x_hbm.at[pi // c, pi % c], bufs[k], rsems[k])
                reads[k].start()
            if p >= 1:
                k = (p - 1) % 2
                reads[k].wait()
                vals = plsc.load_gather(bufs[k], [sxv, syv])
                plsc.store_scatter(bufs[k], [pxv, pyv], vals)
                pi = base + p - 1
                writes[k] = pltpu.make_async_copy(
                    bufs[k], out_hbm.at[pi // c, pi % c], wsems[k])
                writes[k].start()
        writes[(pw - 1) % 2].wait()
        writes[pw % 2].wait()

    return sc_kernel


def kernel(x):
    b, c, h, w = x.shape
    pool_x, pool_y = _pools(h, w)
    px = jnp.asarray(pool_x[:_NUM])
    py = jnp.asarray(pool_y[:_NUM])
    sx = pool_x[-_NUM:]
    sy = pool_y[-_NUM:]
    idx = jnp.asarray(
        np.stack([pool_x[:_NUM], pool_y[:_NUM], sx, sy]).astype(np.int32))

    p = b * c
    info = plsc.get_sparse_core_info()
    nw = info.num_cores * info.num_subcores
    assert p % nw == 0
    out = _make_sc_kernel(b, c, h, w, p // nw)(x, idx)
    return out, (px, py)
